# Initial kernel scaffold; baseline (speedup 1.0000x reference)
import jax
import jax.numpy as jnp
from jax.experimental import pallas as pl


def kernel(emb, wd, wf, whead2, vec, pe_cat, enc_mask, dec_mask, cross_mask, hmt, src, tgt):
    raise NotImplementedError("write your pallas kernel here")


# speedup vs baseline: 4.1811x; 4.1811x over previous
import functools
import math

import jax
import jax.numpy as jnp
from jax.experimental import pallas as pl
from jax.experimental.pallas import tpu as pltpu

_PAD = 128
_DIM, _NHEAD, _NENC, _NDEC, _DFF = 64, 8, 2, 2, 256
_VOCAB = 16
_NB = 8


def _pack_offsets():
    def pack(entries):
        offs, cur = {}, 0
        for key, w in entries:
            offs[key] = (cur, w)
            cur += -(-w // _PAD) * _PAD
        return offs

    D, F, V = _DIM, _DFF, _VOCAB
    wd_e, wf_e, vec_e = [], [], []
    for l in range(_NENC):
        p = f"enc{l}_"
        wd_e += [(p + 'wq', D), (p + 'wk', D), (p + 'wv', D), (p + 'wo', D),
                 (p + 'w1', F)]
        wf_e += [(p + 'w2', D)]
        vec_e += [(p + 'bq', D), (p + 'bk', D), (p + 'bv', D), (p + 'bo', D),
                  (p + 'ln1_g', D), (p + 'ln1_b', D), (p + 'b1', F),
                  (p + 'b2', D), (p + 'ln2_g', D), (p + 'ln2_b', D)]
    for l in range(_NDEC):
        p = f"dec{l}_"
        wd_e += [(p + 'sa_wq', D), (p + 'sa_wk', D), (p + 'sa_wv', D),
                 (p + 'sa_wo', D), (p + 'ca_wq', D), (p + 'ca_wk', D),
                 (p + 'ca_wv', D), (p + 'ca_wo', D), (p + 'w1', F)]
        wf_e += [(p + 'w2', D)]
        vec_e += [(p + 'sa_bq', D), (p + 'sa_bk', D), (p + 'sa_bv', D),
                  (p + 'sa_bo', D), (p + 'ln1_g', D), (p + 'ln1_b', D),
                  (p + 'ca_bq', D), (p + 'ca_bk', D), (p + 'ca_bv', D),
                  (p + 'ca_bo', D), (p + 'ln2_g', D), (p + 'ln2_b', D),
                  (p + 'b1', F), (p + 'b2', D), (p + 'ln3_g', D),
                  (p + 'ln3_b', D)]
    wd_e += [('head_w1p', V), ('head_w1v', D)]
    vec_e += [('enc_norm_g', D), ('enc_norm_b', D),
              ('dec_norm_g', D), ('dec_norm_b', D),
              ('head_b1p', V), ('head_b1v', D), ('head_b2c', _PAD)]
    return pack(wd_e), pack(wf_e), pack(vec_e)


def _batch_kernel(src_ref, tgt_ref, emb_ref, pe_s_ref, pe_t_ref, hmt_ref,
                  wd_ref, wf_ref, wh2_ref, vec_ref, out_ref, *,
                  offs):
    wd_off, wf_off, vec_off = offs
    D, V, H, PAD = _DIM, _VOCAB, _NHEAD, _PAD
    eps = 1e-5
    bf16 = jnp.bfloat16
    NB = _NB
    Ss = src_ref.shape[0] // NB
    St = tgt_ref.shape[0] // NB
    scale = 1.0 / math.sqrt(D // H)

    def wd_w(key, width=None):
        s0, w = wd_off[key]
        return wd_ref[:, s0:s0 + (width if width is not None else w)]

    def wf_w(key):
        s0, w = wf_off[key]
        return wf_ref[:, s0:s0 + w]

    def vb(key, width=None):
        s0, w = vec_off[key]
        return vec_ref[:, s0:s0 + (width if width is not None else w)]

    def mm(x, w):
        return jnp.dot(x.astype(bf16), w, preferred_element_type=jnp.float32)

    def ln(x, g, b):
        mu = jnp.mean(x, axis=-1, keepdims=True)
        var = jnp.mean(jnp.square(x - mu), axis=-1, keepdims=True)
        return (x - mu) * jax.lax.rsqrt(var + eps) * g + b

    hmt = hmt_ref[...]

    def attention(q, k, v, mask):
        Sq = q.shape[0]
        qh = jnp.concatenate([q * hmt[h:h + 1, :] for h in range(H)], axis=0)
        s = jax.lax.dot_general(
            qh.astype(bf16), k.astype(bf16),
            (((1,), (1,)), ((), ())),
            preferred_element_type=jnp.float32) * scale
        if mask is not None:
            s = s + mask
        m = jnp.max(s, axis=-1, keepdims=True)
        e = jnp.exp(s - m)
        p = e * pl.reciprocal(jnp.sum(e, axis=-1, keepdims=True), approx=True)
        o = jnp.dot(p.astype(bf16), v.astype(bf16),
                    preferred_element_type=jnp.float32)
        acc = o[0:Sq, :] * hmt[0:1, :]
        for h in range(1, H):
            acc = acc + o[h * Sq:(h + 1) * Sq, :] * hmt[h:h + 1, :]
        return acc

    def attention_nb(q, k, v, mask, Sq, Sk):
        outs = [attention(q[b * Sq:(b + 1) * Sq],
                          k[b * Sk:(b + 1) * Sk],
                          v[b * Sk:(b + 1) * Sk], mask) for b in range(NB)]
        return jnp.concatenate(outs, axis=0)

    iota_s = jax.lax.broadcasted_iota(jnp.int32, (NB * Ss, V), 1)
    iota_t = jax.lax.broadcasted_iota(jnp.int32, (NB * St, V), 1)
    oh_s = jnp.where(src_ref[...] == iota_s, 1.0, 0.0).astype(bf16)
    oh_t = jnp.where(tgt_ref[...] == iota_t, 1.0, 0.0).astype(bf16)
    pe_s = jnp.concatenate([pe_s_ref[...]] * NB, axis=0)
    pe_t = jnp.concatenate([pe_t_ref[...]] * NB, axis=0)
    x_src = jnp.dot(oh_s, emb_ref[0:V, :],
                    preferred_element_type=jnp.float32) + pe_s
    x_tgt = jnp.dot(oh_t, emb_ref[V:2 * V, :],
                    preferred_element_type=jnp.float32) + pe_t

    qpos = jax.lax.broadcasted_iota(jnp.int32, (St, St), 0)
    kpos = jax.lax.broadcasted_iota(jnp.int32, (St, St), 1)
    causal_1h = jnp.where(kpos <= qpos, 0.0, -1e30).astype(jnp.float32)
    causal = jnp.concatenate([causal_1h] * H, axis=0)

    mem = x_src
    for l in range(_NENC):
        p = f"enc{l}_"
        qkv = mm(mem, wd_w(p + 'wq', 3 * PAD))
        q = qkv[:, 0:D] + vb(p + 'bq')
        k = qkv[:, PAD:PAD + D] + vb(p + 'bk')
        v = qkv[:, 2 * PAD:2 * PAD + D] + vb(p + 'bv')
        a = mm(attention_nb(q, k, v, None, Ss, Ss), wd_w(p + 'wo')) + vb(p + 'bo')
        mem = ln(mem + a, vb(p + 'ln1_g'), vb(p + 'ln1_b'))
        h = jnp.maximum(mm(mem, wd_w(p + 'w1')) + vb(p + 'b1'), 0.0)
        h = mm(h, wf_w(p + 'w2')) + vb(p + 'b2')
        mem = ln(mem + h, vb(p + 'ln2_g'), vb(p + 'ln2_b'))
    mem = ln(mem, vb('enc_norm_g'), vb('enc_norm_b'))

    out = x_tgt
    for l in range(_NDEC):
        p = f"dec{l}_"
        qkv = mm(out, wd_w(p + 'sa_wq', 3 * PAD))
        q = qkv[:, 0:D] + vb(p + 'sa_bq')
        k = qkv[:, PAD:PAD + D] + vb(p + 'sa_bk')
        v = qkv[:, 2 * PAD:2 * PAD + D] + vb(p + 'sa_bv')
        a = mm(attention_nb(q, k, v, causal, St, St), wd_w(p + 'sa_wo')) + vb(p + 'sa_bo')
        out = ln(out + a, vb(p + 'ln1_g'), vb(p + 'ln1_b'))

        q = mm(out, wd_w(p + 'ca_wq')) + vb(p + 'ca_bq')
        kv = mm(mem, wd_w(p + 'ca_wk', 2 * PAD))
        k = kv[:, 0:D] + vb(p + 'ca_bk')
        v = kv[:, PAD:PAD + D] + vb(p + 'ca_bv')
        a = mm(attention_nb(q, k, v, None, St, Ss), wd_w(p + 'ca_wo')) + vb(p + 'ca_bo')
        out = ln(out + a, vb(p + 'ln2_g'), vb(p + 'ln2_b'))

        h = jnp.maximum(mm(out, wd_w(p + 'w1')) + vb(p + 'b1'), 0.0)
        h = mm(h, wf_w(p + 'w2')) + vb(p + 'b2')
        out = ln(out + h, vb(p + 'ln3_g'), vb(p + 'ln3_b'))
    out = ln(out, vb('dec_norm_g'), vb('dec_norm_b'))

    hidden = jnp.maximum(
        mm(out, wd_w('head_w1p', 2 * PAD)) + vb('head_b1p', 2 * PAD), 0.0)
    out_ref[...] = jnp.dot(hidden.astype(bf16), wh2_ref[...],
                           preferred_element_type=jnp.float32) + vb('head_b2c')


def kernel(emb, wd, wf, whead2, vec, pe_cat, enc_mask, dec_mask, cross_mask,
           hmt, src, tgt):
    B, Ss = src.shape
    _, St = tgt.shape
    Ms, Mt = B * Ss, B * St
    V, PAD = _VOCAB, _PAD

    src_ids = src.reshape(Ms, 1).astype(jnp.int32)
    tgt_ids = tgt.reshape(Mt, 1).astype(jnp.int32)
    pe_s = pe_cat[0:Ss]
    pe_t = pe_cat[Ms:Ms + St]

    body = functools.partial(_batch_kernel, offs=_pack_offsets())

    def const_spec(x):
        return pl.BlockSpec(x.shape, lambda i, nd=x.ndim: (0,) * nd)

    out = pl.pallas_call(
        body,
        out_shape=jax.ShapeDtypeStruct((Mt, PAD), jnp.float32),
        grid=(B // _NB,),
        in_specs=[
            pl.BlockSpec((_NB * Ss, 1), lambda i: (i, 0)),
            pl.BlockSpec((_NB * St, 1), lambda i: (i, 0)),
            const_spec(emb),
            const_spec(pe_s),
            const_spec(pe_t),
            const_spec(hmt),
            const_spec(wd),
            const_spec(wf),
            const_spec(whead2),
            const_spec(vec),
        ],
        out_specs=pl.BlockSpec((_NB * St, PAD), lambda i: (i, 0)),
        compiler_params=pltpu.CompilerParams(
            dimension_semantics=("parallel",)),
    )(src_ids, tgt_ids, emb, pe_s, pe_t, hmt, wd, wf, whead2, vec)

    policy = out[:, :V].reshape(B, St, V)
    value = out[:, V:V + 1].reshape(B, St, 1)
    return policy, value


# speedup vs baseline: 4.1869x; 1.0014x over previous
import functools
import math

import jax
import jax.numpy as jnp
from jax.experimental import pallas as pl
from jax.experimental.pallas import tpu as pltpu

_PAD = 128
_DIM, _NHEAD, _NENC, _NDEC, _DFF = 64, 8, 2, 2, 256
_VOCAB = 16
_NB = 8


def _pack_offsets():
    def pack(entries):
        offs, cur = {}, 0
        for key, w in entries:
            offs[key] = (cur, w)
            cur += -(-w // _PAD) * _PAD
        return offs

    D, F, V = _DIM, _DFF, _VOCAB
    wd_e, wf_e, vec_e = [], [], []
    for l in range(_NENC):
        p = f"enc{l}_"
        wd_e += [(p + 'wq', D), (p + 'wk', D), (p + 'wv', D), (p + 'wo', D),
                 (p + 'w1', F)]
        wf_e += [(p + 'w2', D)]
        vec_e += [(p + 'bq', D), (p + 'bk', D), (p + 'bv', D), (p + 'bo', D),
                  (p + 'ln1_g', D), (p + 'ln1_b', D), (p + 'b1', F),
                  (p + 'b2', D), (p + 'ln2_g', D), (p + 'ln2_b', D)]
    for l in range(_NDEC):
        p = f"dec{l}_"
        wd_e += [(p + 'sa_wq', D), (p + 'sa_wk', D), (p + 'sa_wv', D),
                 (p + 'sa_wo', D), (p + 'ca_wq', D), (p + 'ca_wk', D),
                 (p + 'ca_wv', D), (p + 'ca_wo', D), (p + 'w1', F)]
        wf_e += [(p + 'w2', D)]
        vec_e += [(p + 'sa_bq', D), (p + 'sa_bk', D), (p + 'sa_bv', D),
                  (p + 'sa_bo', D), (p + 'ln1_g', D), (p + 'ln1_b', D),
                  (p + 'ca_bq', D), (p + 'ca_bk', D), (p + 'ca_bv', D),
                  (p + 'ca_bo', D), (p + 'ln2_g', D), (p + 'ln2_b', D),
                  (p + 'b1', F), (p + 'b2', D), (p + 'ln3_g', D),
                  (p + 'ln3_b', D)]
    wd_e += [('head_w1p', V), ('head_w1v', D)]
    vec_e += [('enc_norm_g', D), ('enc_norm_b', D),
              ('dec_norm_g', D), ('dec_norm_b', D),
              ('head_b1p', V), ('head_b1v', D), ('head_b2c', _PAD)]
    return pack(wd_e), pack(wf_e), pack(vec_e)


def _batch_kernel(src_ref, tgt_ref, emb_ref, pe_s_ref, pe_t_ref, hmt_ref,
                  wd_ref, wf_ref, wh2_ref, vec_ref, pol_ref, val_ref, *,
                  offs):
    wd_off, wf_off, vec_off = offs
    D, V, H, PAD = _DIM, _VOCAB, _NHEAD, _PAD
    eps = 1e-5
    bf16 = jnp.bfloat16
    NB = _NB
    Ss = src_ref.shape[0] // NB
    St = tgt_ref.shape[0] // NB
    scale = 1.0 / math.sqrt(D // H)

    def wd_w(key, width=None):
        s0, w = wd_off[key]
        return wd_ref[:, s0:s0 + (width if width is not None else w)]

    def wf_w(key):
        s0, w = wf_off[key]
        return wf_ref[:, s0:s0 + w]

    def vb(key, width=None):
        s0, w = vec_off[key]
        return vec_ref[:, s0:s0 + (width if width is not None else w)]

    def mm(x, w):
        return jnp.dot(x.astype(bf16), w, preferred_element_type=jnp.float32)

    def ln(x, g, b):
        mu = jnp.mean(x, axis=-1, keepdims=True)
        var = jnp.mean(jnp.square(x - mu), axis=-1, keepdims=True)
        return (x - mu) * jax.lax.rsqrt(var + eps) * g + b

    hmt = hmt_ref[...]

    def attention(q, k, v, mask):
        Sq = q.shape[0]
        qh = jnp.concatenate([q * hmt[h:h + 1, :] for h in range(H)], axis=0)
        s = jax.lax.dot_general(
            qh.astype(bf16), k.astype(bf16),
            (((1,), (1,)), ((), ())),
            preferred_element_type=jnp.float32) * scale
        if mask is not None:
            s = s + mask
        m = jnp.max(s, axis=-1, keepdims=True)
        e = jnp.exp(s - m)
        p = e * pl.reciprocal(jnp.sum(e, axis=-1, keepdims=True), approx=True)
        o = jnp.dot(p.astype(bf16), v.astype(bf16),
                    preferred_element_type=jnp.float32)
        acc = o[0:Sq, :] * hmt[0:1, :]
        for h in range(1, H):
            acc = acc + o[h * Sq:(h + 1) * Sq, :] * hmt[h:h + 1, :]
        return acc

    def attention_nb(q, k, v, mask, Sq, Sk):
        outs = [attention(q[b * Sq:(b + 1) * Sq],
                          k[b * Sk:(b + 1) * Sk],
                          v[b * Sk:(b + 1) * Sk], mask) for b in range(NB)]
        return jnp.concatenate(outs, axis=0)

    iota_s = jax.lax.broadcasted_iota(jnp.int32, (NB * Ss, V), 1)
    iota_t = jax.lax.broadcasted_iota(jnp.int32, (NB * St, V), 1)
    oh_s = jnp.where(src_ref[...] == iota_s, 1.0, 0.0).astype(bf16)
    oh_t = jnp.where(tgt_ref[...] == iota_t, 1.0, 0.0).astype(bf16)
    pe_s = jnp.concatenate([pe_s_ref[...]] * NB, axis=0)
    pe_t = jnp.concatenate([pe_t_ref[...]] * NB, axis=0)
    x_src = jnp.dot(oh_s, emb_ref[0:V, :],
                    preferred_element_type=jnp.float32) + pe_s
    x_tgt = jnp.dot(oh_t, emb_ref[V:2 * V, :],
                    preferred_element_type=jnp.float32) + pe_t

    qpos = jax.lax.broadcasted_iota(jnp.int32, (St, St), 0)
    kpos = jax.lax.broadcasted_iota(jnp.int32, (St, St), 1)
    causal_1h = jnp.where(kpos <= qpos, 0.0, -1e30).astype(jnp.float32)
    causal = jnp.concatenate([causal_1h] * H, axis=0)

    mem = x_src
    for l in range(_NENC):
        p = f"enc{l}_"
        qkv = mm(mem, wd_w(p + 'wq', 3 * PAD))
        q = qkv[:, 0:D] + vb(p + 'bq')
        k = qkv[:, PAD:PAD + D] + vb(p + 'bk')
        v = qkv[:, 2 * PAD:2 * PAD + D] + vb(p + 'bv')
        a = mm(attention_nb(q, k, v, None, Ss, Ss), wd_w(p + 'wo')) + vb(p + 'bo')
        mem = ln(mem + a, vb(p + 'ln1_g'), vb(p + 'ln1_b'))
        h = jnp.maximum(mm(mem, wd_w(p + 'w1')) + vb(p + 'b1'), 0.0)
        h = mm(h, wf_w(p + 'w2')) + vb(p + 'b2')
        mem = ln(mem + h, vb(p + 'ln2_g'), vb(p + 'ln2_b'))
    mem = ln(mem, vb('enc_norm_g'), vb('enc_norm_b'))

    out = x_tgt
    for l in range(_NDEC):
        p = f"dec{l}_"
        qkv = mm(out, wd_w(p + 'sa_wq', 3 * PAD))
        q = qkv[:, 0:D] + vb(p + 'sa_bq')
        k = qkv[:, PAD:PAD + D] + vb(p + 'sa_bk')
        v = qkv[:, 2 * PAD:2 * PAD + D] + vb(p + 'sa_bv')
        a = mm(attention_nb(q, k, v, causal, St, St), wd_w(p + 'sa_wo')) + vb(p + 'sa_bo')
        out = ln(out + a, vb(p + 'ln1_g'), vb(p + 'ln1_b'))

        q = mm(out, wd_w(p + 'ca_wq')) + vb(p + 'ca_bq')
        kv = mm(mem, wd_w(p + 'ca_wk', 2 * PAD))
        k = kv[:, 0:D] + vb(p + 'ca_bk')
        v = kv[:, PAD:PAD + D] + vb(p + 'ca_bv')
        a = mm(attention_nb(q, k, v, None, St, Ss), wd_w(p + 'ca_wo')) + vb(p + 'ca_bo')
        out = ln(out + a, vb(p + 'ln2_g'), vb(p + 'ln2_b'))

        h = jnp.maximum(mm(out, wd_w(p + 'w1')) + vb(p + 'b1'), 0.0)
        h = mm(h, wf_w(p + 'w2')) + vb(p + 'b2')
        out = ln(out + h, vb(p + 'ln3_g'), vb(p + 'ln3_b'))
    out = ln(out, vb('dec_norm_g'), vb('dec_norm_b'))

    hidden = jnp.maximum(
        mm(out, wd_w('head_w1p', 2 * PAD)) + vb('head_b1p', 2 * PAD), 0.0)
    head = jnp.dot(hidden.astype(bf16), wh2_ref[...],
                   preferred_element_type=jnp.float32) + vb('head_b2c')
    pol_ref[...] = head[:, 0:V]
    val_ref[...] = head[:, V:V + 1]


def kernel(emb, wd, wf, whead2, vec, pe_cat, enc_mask, dec_mask, cross_mask,
           hmt, src, tgt):
    B, Ss = src.shape
    _, St = tgt.shape
    Ms, Mt = B * Ss, B * St
    V, PAD = _VOCAB, _PAD

    src_ids = src.reshape(Ms, 1).astype(jnp.int32)
    tgt_ids = tgt.reshape(Mt, 1).astype(jnp.int32)
    pe_s = pe_cat[0:Ss]
    pe_t = pe_cat[Ms:Ms + St]

    body = functools.partial(_batch_kernel, offs=_pack_offsets())

    def const_spec(x):
        return pl.BlockSpec(x.shape, lambda i, nd=x.ndim: (0,) * nd)

    pol, val = pl.pallas_call(
        body,
        out_shape=[jax.ShapeDtypeStruct((Mt, V), jnp.float32),
                   jax.ShapeDtypeStruct((Mt, 1), jnp.float32)],
        grid=(B // _NB,),
        in_specs=[
            pl.BlockSpec((_NB * Ss, 1), lambda i: (i, 0)),
            pl.BlockSpec((_NB * St, 1), lambda i: (i, 0)),
            const_spec(emb),
            const_spec(pe_s),
            const_spec(pe_t),
            const_spec(hmt),
            const_spec(wd),
            const_spec(wf),
            const_spec(whead2),
            const_spec(vec),
        ],
        out_specs=[pl.BlockSpec((_NB * St, V), lambda i: (i, 0)),
                   pl.BlockSpec((_NB * St, 1), lambda i: (i, 0))],
        compiler_params=pltpu.CompilerParams(
            dimension_semantics=("parallel",)),
    )(src_ids, tgt_ids, emb, pe_s, pe_t, hmt, wd, wf, whead2, vec)

    return pol.reshape(B, St, V), val.reshape(B, St, 1)


# speedup vs baseline: 4.5747x; 1.0926x over previous
import functools
import math

import jax
import jax.numpy as jnp
from jax.experimental import pallas as pl
from jax.experimental.pallas import tpu as pltpu

_PAD = 128
_DIM, _NHEAD, _NENC, _NDEC, _DFF = 64, 8, 2, 2, 256
_VOCAB = 16
_NB = 16


def _pack_offsets():
    def pack(entries):
        offs, cur = {}, 0
        for key, w in entries:
            offs[key] = (cur, w)
            cur += -(-w // _PAD) * _PAD
        return offs

    D, F, V = _DIM, _DFF, _VOCAB
    wd_e, wf_e, vec_e = [], [], []
    for l in range(_NENC):
        p = f"enc{l}_"
        wd_e += [(p + 'wq', D), (p + 'wk', D), (p + 'wv', D), (p + 'wo', D),
                 (p + 'w1', F)]
        wf_e += [(p + 'w2', D)]
        vec_e += [(p + 'bq', D), (p + 'bk', D), (p + 'bv', D), (p + 'bo', D),
                  (p + 'ln1_g', D), (p + 'ln1_b', D), (p + 'b1', F),
                  (p + 'b2', D), (p + 'ln2_g', D), (p + 'ln2_b', D)]
    for l in range(_NDEC):
        p = f"dec{l}_"
        wd_e += [(p + 'sa_wq', D), (p + 'sa_wk', D), (p + 'sa_wv', D),
                 (p + 'sa_wo', D), (p + 'ca_wq', D), (p + 'ca_wk', D),
                 (p + 'ca_wv', D), (p + 'ca_wo', D), (p + 'w1', F)]
        wf_e += [(p + 'w2', D)]
        vec_e += [(p + 'sa_bq', D), (p + 'sa_bk', D), (p + 'sa_bv', D),
                  (p + 'sa_bo', D), (p + 'ln1_g', D), (p + 'ln1_b', D),
                  (p + 'ca_bq', D), (p + 'ca_bk', D), (p + 'ca_bv', D),
                  (p + 'ca_bo', D), (p + 'ln2_g', D), (p + 'ln2_b', D),
                  (p + 'b1', F), (p + 'b2', D), (p + 'ln3_g', D),
                  (p + 'ln3_b', D)]
    wd_e += [('head_w1p', V), ('head_w1v', D)]
    vec_e += [('enc_norm_g', D), ('enc_norm_b', D),
              ('dec_norm_g', D), ('dec_norm_b', D),
              ('head_b1p', V), ('head_b1v', D), ('head_b2c', _PAD)]
    return pack(wd_e), pack(wf_e), pack(vec_e)


def _batch_kernel(src_ref, tgt_ref, emb_ref, pe_s_ref, pe_t_ref, hmt_ref,
                  wd_ref, wf_ref, wh2_ref, vec_ref, pol_ref, val_ref, *,
                  offs):
    wd_off, wf_off, vec_off = offs
    D, V, H, PAD = _DIM, _VOCAB, _NHEAD, _PAD
    eps = 1e-5
    bf16 = jnp.bfloat16
    NB = _NB
    Ss = src_ref.shape[0] // NB
    St = tgt_ref.shape[0] // NB
    scale = 1.0 / math.sqrt(D // H)

    def wd_w(key, width=None):
        s0, w = wd_off[key]
        return wd_ref[:, s0:s0 + (width if width is not None else w)]

    def wf_w(key):
        s0, w = wf_off[key]
        return wf_ref[:, s0:s0 + w]

    def vb(key, width=None):
        s0, w = vec_off[key]
        return vec_ref[:, s0:s0 + (width if width is not None else w)]

    def mm(x, w):
        return jnp.dot(x.astype(bf16), w, preferred_element_type=jnp.float32)

    def ln(x, g, b):
        mu = jnp.mean(x, axis=-1, keepdims=True)
        var = jnp.mean(jnp.square(x - mu), axis=-1, keepdims=True)
        return (x - mu) * jax.lax.rsqrt(var + eps) * g + b

    hmt = hmt_ref[...]

    def attention(q, k, v, mask):
        Sq = q.shape[0]
        qh = jnp.concatenate([q * hmt[h:h + 1, :] for h in range(H)], axis=0)
        s = jax.lax.dot_general(
            qh.astype(bf16), k.astype(bf16),
            (((1,), (1,)), ((), ())),
            preferred_element_type=jnp.float32) * scale
        if mask is not None:
            s = s + mask
        m = jnp.max(s, axis=-1, keepdims=True)
        e = jnp.exp(s - m)
        p = e * pl.reciprocal(jnp.sum(e, axis=-1, keepdims=True), approx=True)
        o = jnp.dot(p.astype(bf16), v.astype(bf16),
                    preferred_element_type=jnp.float32)
        acc = o[0:Sq, :] * hmt[0:1, :]
        for h in range(1, H):
            acc = acc + o[h * Sq:(h + 1) * Sq, :] * hmt[h:h + 1, :]
        return acc

    def attention_nb(q, k, v, mask, Sq, Sk):
        outs = [attention(q[b * Sq:(b + 1) * Sq],
                          k[b * Sk:(b + 1) * Sk],
                          v[b * Sk:(b + 1) * Sk], mask) for b in range(NB)]
        return jnp.concatenate(outs, axis=0)

    iota_s = jax.lax.broadcasted_iota(jnp.int32, (NB * Ss, V), 1)
    iota_t = jax.lax.broadcasted_iota(jnp.int32, (NB * St, V), 1)
    oh_s = jnp.where(src_ref[...] == iota_s, 1.0, 0.0).astype(bf16)
    oh_t = jnp.where(tgt_ref[...] == iota_t, 1.0, 0.0).astype(bf16)
    pe_s = jnp.concatenate([pe_s_ref[...]] * NB, axis=0)
    pe_t = jnp.concatenate([pe_t_ref[...]] * NB, axis=0)
    x_src = jnp.dot(oh_s, emb_ref[0:V, :],
                    preferred_element_type=jnp.float32) + pe_s
    x_tgt = jnp.dot(oh_t, emb_ref[V:2 * V, :],
                    preferred_element_type=jnp.float32) + pe_t

    qpos = jax.lax.broadcasted_iota(jnp.int32, (St, St), 0)
    kpos = jax.lax.broadcasted_iota(jnp.int32, (St, St), 1)
    causal_1h = jnp.where(kpos <= qpos, 0.0, -1e30).astype(jnp.float32)
    causal = jnp.concatenate([causal_1h] * H, axis=0)

    mem = x_src
    for l in range(_NENC):
        p = f"enc{l}_"
        qkv = mm(mem, wd_w(p + 'wq', 3 * PAD))
        q = qkv[:, 0:D] + vb(p + 'bq')
        k = qkv[:, PAD:PAD + D] + vb(p + 'bk')
        v = qkv[:, 2 * PAD:2 * PAD + D] + vb(p + 'bv')
        a = mm(attention_nb(q, k, v, None, Ss, Ss), wd_w(p + 'wo')) + vb(p + 'bo')
        mem = ln(mem + a, vb(p + 'ln1_g'), vb(p + 'ln1_b'))
        h = jnp.maximum(mm(mem, wd_w(p + 'w1')) + vb(p + 'b1'), 0.0)
        h = mm(h, wf_w(p + 'w2')) + vb(p + 'b2')
        mem = ln(mem + h, vb(p + 'ln2_g'), vb(p + 'ln2_b'))
    mem = ln(mem, vb('enc_norm_g'), vb('enc_norm_b'))

    out = x_tgt
    for l in range(_NDEC):
        p = f"dec{l}_"
        qkv = mm(out, wd_w(p + 'sa_wq', 3 * PAD))
        q = qkv[:, 0:D] + vb(p + 'sa_bq')
        k = qkv[:, PAD:PAD + D] + vb(p + 'sa_bk')
        v = qkv[:, 2 * PAD:2 * PAD + D] + vb(p + 'sa_bv')
        a = mm(attention_nb(q, k, v, causal, St, St), wd_w(p + 'sa_wo')) + vb(p + 'sa_bo')
        out = ln(out + a, vb(p + 'ln1_g'), vb(p + 'ln1_b'))

        q = mm(out, wd_w(p + 'ca_wq')) + vb(p + 'ca_bq')
        kv = mm(mem, wd_w(p + 'ca_wk', 2 * PAD))
        k = kv[:, 0:D] + vb(p + 'ca_bk')
        v = kv[:, PAD:PAD + D] + vb(p + 'ca_bv')
        a = mm(attention_nb(q, k, v, None, St, Ss), wd_w(p + 'ca_wo')) + vb(p + 'ca_bo')
        out = ln(out + a, vb(p + 'ln2_g'), vb(p + 'ln2_b'))

        h = jnp.maximum(mm(out, wd_w(p + 'w1')) + vb(p + 'b1'), 0.0)
        h = mm(h, wf_w(p + 'w2')) + vb(p + 'b2')
        out = ln(out + h, vb(p + 'ln3_g'), vb(p + 'ln3_b'))
    out = ln(out, vb('dec_norm_g'), vb('dec_norm_b'))

    hidden = jnp.maximum(
        mm(out, wd_w('head_w1p', 2 * PAD)) + vb('head_b1p', 2 * PAD), 0.0)
    head = jnp.dot(hidden.astype(bf16), wh2_ref[...],
                   preferred_element_type=jnp.float32) + vb('head_b2c')
    pol_ref[...] = head[:, 0:V]
    val_ref[...] = head[:, V:V + 1]


def kernel(emb, wd, wf, whead2, vec, pe_cat, enc_mask, dec_mask, cross_mask,
           hmt, src, tgt):
    B, Ss = src.shape
    _, St = tgt.shape
    Ms, Mt = B * Ss, B * St
    V, PAD = _VOCAB, _PAD

    src_ids = src.reshape(Ms, 1).astype(jnp.int32)
    tgt_ids = tgt.reshape(Mt, 1).astype(jnp.int32)
    pe_s = pe_cat[0:Ss]
    pe_t = pe_cat[Ms:Ms + St]

    body = functools.partial(_batch_kernel, offs=_pack_offsets())

    def const_spec(x):
        return pl.BlockSpec(x.shape, lambda i, nd=x.ndim: (0,) * nd)

    pol, val = pl.pallas_call(
        body,
        out_shape=[jax.ShapeDtypeStruct((Mt, V), jnp.float32),
                   jax.ShapeDtypeStruct((Mt, 1), jnp.float32)],
        grid=(B // _NB,),
        in_specs=[
            pl.BlockSpec((_NB * Ss, 1), lambda i: (i, 0)),
            pl.BlockSpec((_NB * St, 1), lambda i: (i, 0)),
            const_spec(emb),
            const_spec(pe_s),
            const_spec(pe_t),
            const_spec(hmt),
            const_spec(wd),
            const_spec(wf),
            const_spec(whead2),
            const_spec(vec),
        ],
        out_specs=[pl.BlockSpec((_NB * St, V), lambda i: (i, 0)),
                   pl.BlockSpec((_NB * St, 1), lambda i: (i, 0))],
        compiler_params=pltpu.CompilerParams(
            dimension_semantics=("parallel",)),
    )(src_ids, tgt_ids, emb, pe_s, pe_t, hmt, wd, wf, whead2, vec)

    return pol.reshape(B, St, V), val.reshape(B, St, 1)
